# baseline (speedup 1.0000x reference)
import jax
import jax.numpy as jnp
from jax import lax
from jax.experimental import pallas as pl
from jax.experimental.pallas import tpu as pltpu
from jax.experimental.pallas import tpu_sc as plsc

_B = 16384
_D = 64
_NW = 32
_BPW = _B // _NW
_CHUNK = 128
_NCH = _BPW // _CHUNK
_GPC = _CHUNK // 16
_CB = 16384

_SLAB = 256
_NBUF = 4
_NSLAB = 56
_CPT = _SLAB * _NSLAB
_SC_COLS = _CPT * _NW
_TC_FIRST = _SC_COLS // _CB
_NBLK_U = (1000000 - _SC_COLS + _CB - 1) // _CB
_SU_ROWS = _SC_COLS // 128 + _NBLK_U * (_CB // 128)


def _sc_matvec_body(tT, w_hbm, out_hbm, *rest):
    slabs = rest[:_NBUF]
    wv = rest[_NBUF]
    obs = rest[_NBUF + 1:2 * _NBUF + 1]
    sems = rest[2 * _NBUF + 1:3 * _NBUF + 1]
    sem_o = rest[3 * _NBUF + 1]

    cid = lax.axis_index("c")
    sid = lax.axis_index("s")
    wid = cid * 16 + sid
    c0 = wid * _CPT

    pltpu.sync_copy(w_hbm, wv)
    w_vecs = [wv[pl.ds(16 * t, 16)] for t in range(_D // 16)]

    def fire(s, buf, sem):
        return pltpu.async_copy(tT.at[:, pl.ds(c0 + s * _SLAB, _SLAB)],
                                buf, sem)

    def compute(buf, ob, s, p):
        @pl.when(p >= 1)
        def _():
            pltpu.make_async_copy(
                ob, out_hbm.at[pl.ds(c0, _SLAB)], sem_o).wait()

        def gbody(g, carry):
            cols = pl.ds(g * 16, 16)
            acc0 = jnp.zeros((16,), jnp.float32)
            acc1 = jnp.zeros((16,), jnp.float32)
            acc2 = jnp.zeros((16,), jnp.float32)
            acc3 = jnp.zeros((16,), jnp.float32)
            for d in range(_D):
                v = buf[d, cols] * w_vecs[d // 16][d % 16]
                if d % 4 == 0:
                    acc0 = acc0 + v
                elif d % 4 == 1:
                    acc1 = acc1 + v
                elif d % 4 == 2:
                    acc2 = acc2 + v
                else:
                    acc3 = acc3 + v
            ob[cols] = (acc0 + acc1) + (acc2 + acc3)
            return carry

        lax.fori_loop(0, _SLAB // 16, gbody, 0)
        pltpu.async_copy(ob, out_hbm.at[pl.ds(c0 + s * _SLAB, _SLAB)], sem_o)

    for q in range(_NBUF):
        fire(q, slabs[q], sems[q])

    def pbody(p, carry):
        s0 = _NBUF * p
        for q in range(_NBUF):
            pltpu.make_async_copy(tT.at[:, pl.ds(0, _SLAB)],
                                  slabs[q], sems[q]).wait()
            compute(slabs[q], obs[q], s0 + q, p)
            fire(s0 + q + _NBUF, slabs[q], sems[q])
        return carry

    lax.fori_loop(0, _NSLAB // _NBUF, pbody, 0)
    for q in range(_NBUF):
        pltpu.make_async_copy(tT.at[:, pl.ds(0, _SLAB)],
                              slabs[q], sems[q]).wait()
        pltpu.make_async_copy(obs[q], out_hbm.at[pl.ds(c0, _SLAB)],
                              sem_o).wait()


def _sc_matvec(tT, wflat):
    run = pl.kernel(
        _sc_matvec_body,
        out_type=jax.ShapeDtypeStruct((_SU_ROWS * 128,), jnp.float32),
        mesh=plsc.VectorSubcoreMesh(core_axis_name="c", subcore_axis_name="s"),
        compiler_params=pltpu.CompilerParams(needs_layout_passes=False),
        scratch_types=(
            [pltpu.VMEM((_D, _SLAB), jnp.float32) for _ in range(_NBUF)]
            + [pltpu.VMEM((_D,), jnp.float32)]
            + [pltpu.VMEM((_SLAB,), jnp.float32) for _ in range(_NBUF)]
            + [pltpu.SemaphoreType.DMA for _ in range(_NBUF + 1)]
        ),
    )
    return run(tT, wflat)


def _tc_matvec_body(w_ref, x_ref, o_ref):
    x = x_ref[...]
    w = w_ref[...]
    pieces = []
    for s in range(_CB // 128):
        xs = x[:, s * 128:(s + 1) * 128]
        pieces.append(jnp.sum(xs * w, axis=0, keepdims=True))
    o_ref[...] = jnp.concatenate(pieces, axis=0)


def _tc_scores(tT, wcol, first, nblk):
    return pl.pallas_call(
        _tc_matvec_body,
        grid=(nblk,),
        in_specs=[
            pl.BlockSpec((_D, 1), lambda i: (0, 0)),
            pl.BlockSpec((_D, _CB), lambda i: (0, first + i)),
        ],
        out_specs=pl.BlockSpec((_CB // 128, 128), lambda i: (i, 0)),
        out_shape=jax.ShapeDtypeStruct((nblk * (_CB // 128), 128), jnp.float32),
    )(wcol, tT)


def _tc_matvec_into_body(w_ref, x_ref, prev_ref, o_ref):
    del prev_ref
    _tc_matvec_body(w_ref, x_ref, o_ref)


def _tc_scores_into(tT, wcol, first, nblk, prev):
    rpb = _CB // 128
    return pl.pallas_call(
        _tc_matvec_into_body,
        grid=(nblk,),
        in_specs=[
            pl.BlockSpec((_D, 1), lambda i: (0, 0)),
            pl.BlockSpec((_D, _CB), lambda i, first=first: (0, first + i)),
            pl.BlockSpec(memory_space=pl.ANY),
        ],
        out_specs=pl.BlockSpec((rpb, 128), lambda i, first=first: (first + i, 0)),
        out_shape=jax.ShapeDtypeStruct(prev.shape, jnp.float32),
        input_output_aliases={2: 0},
    )(wcol, tT, prev)


def _sc_gather_body(user_hbm, item_hbm, su_hbm, si_hbm, b_hbm, out_hbm,
                    idx_u, idx_i, lane_u, lane_i, bu0, bu1, bi0, bi1,
                    b_v, out_v, sem_u, sem_i):
    cid = lax.axis_index("c")
    sid = lax.axis_index("s")
    wid = cid * 16 + sid
    base = wid * _BPW

    cu0 = pltpu.async_copy(user_hbm.at[pl.ds(base, _BPW)], idx_u, sem_u)
    ci0 = pltpu.async_copy(item_hbm.at[pl.ds(base, _BPW)], idx_i, sem_i)
    cb0 = pltpu.async_copy(b_hbm, b_v, sem_u)
    cu0.wait()
    ci0.wait()
    cb0.wait()

    for t in range(_BPW // 16):
        s = pl.ds(t * 16, 16)
        vu = idx_u[s]
        vi = idx_i[s]
        idx_u[s] = lax.shift_right_logical(vu, 7)
        idx_i[s] = lax.shift_right_logical(vi, 7)
        lane_u[s] = vu & 127
        lane_i[s] = vi & 127

    bus = [bu0, bu1]
    bis = [bi0, bi1]

    def fire(k):
        cu = pltpu.async_copy(
            su_hbm.at[idx_u.at[pl.ds(k * _CHUNK, _CHUNK)]], bus[k % 2], sem_u)
        ci = pltpu.async_copy(
            si_hbm.at[idx_i.at[pl.ds(k * _CHUNK, _CHUNK)]], bis[k % 2], sem_i)
        return cu, ci

    bias = b_v[pl.ds(0, 16)][0]

    pending = fire(0)
    for k in range(_NCH):
        nxt = fire(k + 1) if k + 1 < _NCH else None
        pending[0].wait()
        pending[1].wait()
        pending = nxt
        bu, bi = bus[k % 2], bis[k % 2]

        def group_body(g, carry, k=k, bu=bu, bi=bi):
            jvec = g * 16 + lax.iota(jnp.int32, 16)
            lu = lane_u[pl.ds(k * _CHUNK + g * 16, 16)]
            li = lane_i[pl.ds(k * _CHUNK + g * 16, 16)]
            gu = plsc.load_gather(bu, [jvec, lu])
            gi = plsc.load_gather(bi, [jvec, li])
            out_v[pl.ds(k * _CHUNK + g * 16, 16)] = gu + gi + bias
            return carry

        lax.fori_loop(0, _GPC, group_body, 0)

    pltpu.sync_copy(out_v, out_hbm.at[pl.ds(base, _BPW)])


@jax.jit
def kernel(user, item, user_table, item_table, W, b):
    utT = user_table.T
    itT = item_table.T
    wu_col = W[0, :_D].reshape(_D, 1)
    wi_col = W[0, _D:].reshape(_D, 1)
    wu_flat = W[0, :_D]

    nblk_u = (utT.shape[1] - _SC_COLS + _CB - 1) // _CB
    su_buf = _sc_matvec(utT, wu_flat)
    su = _tc_scores_into(utT, wu_col, _TC_FIRST, nblk_u,
                         su_buf.reshape(-1, 128))
    nblk_i = (itT.shape[1] + _CB - 1) // _CB
    si = _tc_scores(itT, wi_col, 0, nblk_i)

    bpad = jnp.concatenate([b, jnp.zeros((15,), jnp.float32)])

    run = pl.kernel(
        _sc_gather_body,
        out_type=jax.ShapeDtypeStruct((_B,), jnp.float32),
        mesh=plsc.VectorSubcoreMesh(core_axis_name="c", subcore_axis_name="s"),
        compiler_params=pltpu.CompilerParams(needs_layout_passes=False),
        scratch_types=[
            pltpu.VMEM((_BPW,), jnp.int32),
            pltpu.VMEM((_BPW,), jnp.int32),
            pltpu.VMEM((_BPW,), jnp.int32),
            pltpu.VMEM((_BPW,), jnp.int32),
            pltpu.VMEM((_CHUNK, 128), jnp.float32),
            pltpu.VMEM((_CHUNK, 128), jnp.float32),
            pltpu.VMEM((_CHUNK, 128), jnp.float32),
            pltpu.VMEM((_CHUNK, 128), jnp.float32),
            pltpu.VMEM((16,), jnp.float32),
            pltpu.VMEM((_BPW,), jnp.float32),
            pltpu.SemaphoreType.DMA,
            pltpu.SemaphoreType.DMA,
        ],
    )
    y = run(user.astype(jnp.int32), item.astype(jnp.int32), su, si, bpad)
    return y.reshape(_B, 1)


# speedup vs baseline: 1.3066x; 1.3066x over previous
import jax
import jax.numpy as jnp
from jax import lax
from jax.experimental import pallas as pl
from jax.experimental.pallas import tpu as pltpu
from jax.experimental.pallas import tpu_sc as plsc

_B = 16384
_D = 64
_NW = 32
_BPW = _B // _NW
_CHUNK = 128
_NCH = _BPW // _CHUNK
_GPC = _CHUNK // 16
_CB = 16384

_SLAB = 512
_NSLAB = 28
_CPT = _SLAB * _NSLAB
_SC_COLS = _CPT * _NW
_TC_FIRST = _SC_COLS // _CB


def _sc_matvec_body(tT, w_hbm, out_hbm, sl0, sl1, wv, ob0, ob1,
                    sem0, sem1, sem_o):
    cid = lax.axis_index("c")
    sid = lax.axis_index("s")
    wid = cid * 16 + sid
    c0 = wid * _CPT

    pltpu.sync_copy(w_hbm, wv)
    w_vecs = [wv[pl.ds(16 * t, 16)] for t in range(_D // 16)]

    def fire(s, buf, sem):
        return pltpu.async_copy(tT.at[:, pl.ds(c0 + s * _SLAB, _SLAB)],
                                buf, sem)

    def compute(buf, ob, s, p):
        @pl.when(p >= 1)
        def _():
            pltpu.make_async_copy(
                ob, out_hbm.at[pl.ds(c0, _SLAB)], sem_o).wait()

        def gbody(g, carry):
            cols = pl.ds(g * 16, 16)
            acc0 = jnp.zeros((16,), jnp.float32)
            acc1 = jnp.zeros((16,), jnp.float32)
            acc2 = jnp.zeros((16,), jnp.float32)
            acc3 = jnp.zeros((16,), jnp.float32)
            for d in range(_D):
                v = buf[d, cols] * w_vecs[d // 16][d % 16]
                if d % 4 == 0:
                    acc0 = acc0 + v
                elif d % 4 == 1:
                    acc1 = acc1 + v
                elif d % 4 == 2:
                    acc2 = acc2 + v
                else:
                    acc3 = acc3 + v
            ob[cols] = (acc0 + acc1) + (acc2 + acc3)
            return carry

        lax.fori_loop(0, _SLAB // 16, gbody, 0)
        pltpu.async_copy(ob, out_hbm.at[pl.ds(c0 + s * _SLAB, _SLAB)], sem_o)

    fire(0, sl0, sem0)
    fire(1, sl1, sem1)

    def pbody(p, carry):
        s0 = 2 * p
        pltpu.make_async_copy(tT.at[:, pl.ds(0, _SLAB)], sl0, sem0).wait()
        compute(sl0, ob0, s0, p)
        fire(s0 + 2, sl0, sem0)
        pltpu.make_async_copy(tT.at[:, pl.ds(0, _SLAB)], sl1, sem1).wait()
        compute(sl1, ob1, s0 + 1, p)
        fire(s0 + 3, sl1, sem1)
        return carry

    lax.fori_loop(0, _NSLAB // 2, pbody, 0)
    pltpu.make_async_copy(tT.at[:, pl.ds(0, _SLAB)], sl0, sem0).wait()
    pltpu.make_async_copy(tT.at[:, pl.ds(0, _SLAB)], sl1, sem1).wait()
    pltpu.make_async_copy(ob0, out_hbm.at[pl.ds(c0, _SLAB)], sem_o).wait()
    pltpu.make_async_copy(ob1, out_hbm.at[pl.ds(c0, _SLAB)], sem_o).wait()


def _sc_matvec(tT, wflat):
    run = pl.kernel(
        _sc_matvec_body,
        out_type=jax.ShapeDtypeStruct((_SC_COLS,), jnp.float32),
        mesh=plsc.VectorSubcoreMesh(core_axis_name="c", subcore_axis_name="s"),
        compiler_params=pltpu.CompilerParams(needs_layout_passes=False),
        scratch_types=[
            pltpu.VMEM((_D, _SLAB), jnp.float32),
            pltpu.VMEM((_D, _SLAB), jnp.float32),
            pltpu.VMEM((_D,), jnp.float32),
            pltpu.VMEM((_SLAB,), jnp.float32),
            pltpu.VMEM((_SLAB,), jnp.float32),
            pltpu.SemaphoreType.DMA,
            pltpu.SemaphoreType.DMA,
            pltpu.SemaphoreType.DMA,
        ],
    )
    return run(tT, wflat)


def _tc_matvec_body(w_ref, x_ref, o_ref):
    x = x_ref[...]
    w = w_ref[...]
    pieces = []
    for s in range(_CB // 128):
        xs = x[:, s * 128:(s + 1) * 128]
        pieces.append(jnp.sum(xs * w, axis=0, keepdims=True))
    o_ref[...] = jnp.concatenate(pieces, axis=0)


def _tc_scores(tT, wcol, first, nblk):
    return pl.pallas_call(
        _tc_matvec_body,
        grid=(nblk,),
        in_specs=[
            pl.BlockSpec((_D, 1), lambda i: (0, 0)),
            pl.BlockSpec((_D, _CB), lambda i: (0, first + i)),
        ],
        out_specs=pl.BlockSpec((_CB // 128, 128), lambda i: (i, 0)),
        out_shape=jax.ShapeDtypeStruct((nblk * (_CB // 128), 128), jnp.float32),
    )(wcol, tT)


def _sc_gather_body(user_hbm, item_hbm, su_hbm, si_hbm, b_hbm, out_hbm,
                    idx_u, idx_i, lane_u, lane_i, bu0, bu1, bi0, bi1,
                    b_v, out_v, sem_u, sem_i):
    cid = lax.axis_index("c")
    sid = lax.axis_index("s")
    wid = cid * 16 + sid
    base = wid * _BPW

    cu0 = pltpu.async_copy(user_hbm.at[pl.ds(base, _BPW)], idx_u, sem_u)
    ci0 = pltpu.async_copy(item_hbm.at[pl.ds(base, _BPW)], idx_i, sem_i)
    cb0 = pltpu.async_copy(b_hbm, b_v, sem_u)
    cu0.wait()
    ci0.wait()
    cb0.wait()

    for t in range(_BPW // 16):
        s = pl.ds(t * 16, 16)
        vu = idx_u[s]
        vi = idx_i[s]
        idx_u[s] = lax.shift_right_logical(vu, 7)
        idx_i[s] = lax.shift_right_logical(vi, 7)
        lane_u[s] = vu & 127
        lane_i[s] = vi & 127

    bus = [bu0, bu1]
    bis = [bi0, bi1]

    def fire(k):
        cu = pltpu.async_copy(
            su_hbm.at[idx_u.at[pl.ds(k * _CHUNK, _CHUNK)]], bus[k % 2], sem_u)
        ci = pltpu.async_copy(
            si_hbm.at[idx_i.at[pl.ds(k * _CHUNK, _CHUNK)]], bis[k % 2], sem_i)
        return cu, ci

    bias = b_v[pl.ds(0, 16)][0]

    pending = fire(0)
    for k in range(_NCH):
        nxt = fire(k + 1) if k + 1 < _NCH else None
        pending[0].wait()
        pending[1].wait()
        pending = nxt
        bu, bi = bus[k % 2], bis[k % 2]

        def group_body(g, carry, k=k, bu=bu, bi=bi):
            jvec = g * 16 + lax.iota(jnp.int32, 16)
            lu = lane_u[pl.ds(k * _CHUNK + g * 16, 16)]
            li = lane_i[pl.ds(k * _CHUNK + g * 16, 16)]
            gu = plsc.load_gather(bu, [jvec, lu])
            gi = plsc.load_gather(bi, [jvec, li])
            out_v[pl.ds(k * _CHUNK + g * 16, 16)] = gu + gi + bias
            return carry

        lax.fori_loop(0, _GPC, group_body, 0)

    pltpu.sync_copy(out_v, out_hbm.at[pl.ds(base, _BPW)])


@jax.jit
def kernel(user, item, user_table, item_table, W, b):
    utT = user_table.T
    itT = item_table.T
    wu_col = W[0, :_D].reshape(_D, 1)
    wi_col = W[0, _D:].reshape(_D, 1)
    wu_flat = W[0, :_D]

    su_sc = _sc_matvec(utT, wu_flat)
    nblk_u = (utT.shape[1] - _SC_COLS + _CB - 1) // _CB
    su_tc = _tc_scores(utT, wu_col, _TC_FIRST, nblk_u)
    nblk_i = (itT.shape[1] + _CB - 1) // _CB
    si = _tc_scores(itT, wi_col, 0, nblk_i)

    su = jnp.concatenate([su_sc.reshape(_SC_COLS // 128, 128), su_tc],
                         axis=0)

    bpad = jnp.concatenate([b, jnp.zeros((15,), jnp.float32)])

    run = pl.kernel(
        _sc_gather_body,
        out_type=jax.ShapeDtypeStruct((_B,), jnp.float32),
        mesh=plsc.VectorSubcoreMesh(core_axis_name="c", subcore_axis_name="s"),
        compiler_params=pltpu.CompilerParams(needs_layout_passes=False),
        scratch_types=[
            pltpu.VMEM((_BPW,), jnp.int32),
            pltpu.VMEM((_BPW,), jnp.int32),
            pltpu.VMEM((_BPW,), jnp.int32),
            pltpu.VMEM((_BPW,), jnp.int32),
            pltpu.VMEM((_CHUNK, 128), jnp.float32),
            pltpu.VMEM((_CHUNK, 128), jnp.float32),
            pltpu.VMEM((_CHUNK, 128), jnp.float32),
            pltpu.VMEM((_CHUNK, 128), jnp.float32),
            pltpu.VMEM((16,), jnp.float32),
            pltpu.VMEM((_BPW,), jnp.float32),
            pltpu.SemaphoreType.DMA,
            pltpu.SemaphoreType.DMA,
        ],
    )
    y = run(user.astype(jnp.int32), item.astype(jnp.int32), su, si, bpad)
    return y.reshape(_B, 1)
